# baseline (speedup 1.0000x reference)
import functools

import jax
import jax.numpy as jnp
from jax import lax
from jax.experimental import pallas as pl
from jax.experimental.pallas import tpu as pltpu
from jax.experimental.pallas import tpu_sc as plsc

NODE = 1_000_000
MEM = 32
NEV = 16384
TWOB = 2 * NEV
B4 = NEV // 4

NC, NS = 2, 16
NW = NC * NS
G_ROWS = TWOB // NW // 128
R_ROWS = TWOB // NS // 128
R_CHUNK = R_ROWS * 128
WTAB = NODE + 128
TRASH = NODE
ROUNDS = 3

_SC_PARAMS = pltpu.CompilerParams(use_tc_tiling_on_sc=False)


def _sc_mesh():
    return plsc.VectorSubcoreMesh(
        core_axis_name="c", subcore_axis_name="s", num_cores=NC, num_subcores=NS)


@functools.cache
def _make_sc_gather(dtype):
    return pl.kernel(
        _sc_gather_body,
        out_type=jax.ShapeDtypeStruct((TWOB, MEM), dtype),
        mesh=_sc_mesh(),
        scratch_types=[
            pltpu.VMEM((G_ROWS, 128), jnp.int32),
            pltpu.VMEM((G_ROWS * 128, MEM), dtype),
            pltpu.SemaphoreType.DMA,
        ],
        compiler_params=_SC_PARAMS,
    )


def _sc_gather_body(mem_hbm, idx_hbm, out_hbm, idx_v, rows_v, sem):
    wid = lax.axis_index("s") * NC + lax.axis_index("c")
    r0 = wid * G_ROWS
    pltpu.sync_copy(idx_hbm.at[pl.ds(r0, G_ROWS)], idx_v)
    cps = [
        pltpu.async_copy(mem_hbm.at[idx_v.at[b]], rows_v.at[pl.ds(b * 128, 128)], sem)
        for b in range(G_ROWS)
    ]
    for c in cps:
        c.wait()
    pltpu.sync_copy(rows_v, out_hbm.at[pl.ds(r0 * 128, G_ROWS * 128)])


NSPLIT = NODE // 2
LTRASH = NSPLIT
LTAB = NSPLIT + 128
R_CH16 = R_CHUNK // 16


@functools.cache
def _make_sc_resolve():
    return pl.kernel(
        _sc_resolve_body,
        out_type=jax.ShapeDtypeStruct((NC, TWOB), jnp.int32),
        mesh=_sc_mesh(),
        scratch_types=[
            pltpu.VMEM_SHARED((LTAB,), jnp.int32),
            pltpu.VMEM((R_CHUNK,), jnp.int32),
            pltpu.VMEM((R_CHUNK,), jnp.int32),
            pltpu.VMEM((R_CHUNK,), jnp.int32),
            pltpu.VMEM((R_CHUNK,), jnp.int32),
            pltpu.SemaphoreType.DMA,
        ],
        compiler_params=_SC_PARAMS,
    )


def _sc_resolve_body(idx_hbm, j_hbm, out_hbm, wtab_sh,
                     li, jv, wv, tv, sem):
    cid = lax.axis_index("c")
    sid = lax.axis_index("s")
    base = cid * NSPLIT
    q0 = sid * R_CHUNK
    pltpu.sync_copy(idx_hbm.at[pl.ds(q0, R_CHUNK)], li)
    pltpu.sync_copy(j_hbm.at[pl.ds(q0, R_CHUNK)], jv)

    def _localize(i, carry):
        sl = pl.ds(i * 16, 16)
        loc = li[sl] - base
        ok = (loc >= 0) & (loc < NSPLIT)
        li[sl] = jnp.where(ok, loc, jnp.full((16,), LTRASH, jnp.int32))
        return carry

    lax.fori_loop(0, R_CH16, _localize, 0)

    pltpu.async_copy(jv, wtab_sh.at[li], sem).wait()
    plsc.subcore_barrier()

    for _ in range(ROUNDS):
        pltpu.async_copy(wtab_sh.at[li], wv, sem).wait()
        plsc.subcore_barrier()

        def _row(i, carry):
            sl = pl.ds(i * 16, 16)
            keep = (jv[sl] > wv[sl]) & (li[sl] < LTRASH)
            tv[sl] = jnp.where(keep, li[sl], jnp.full((16,), LTRASH, jnp.int32))
            return carry

        lax.fori_loop(0, R_CH16, _row, 0)

        pltpu.async_copy(jv, wtab_sh.at[tv], sem).wait()
        plsc.subcore_barrier()

    pltpu.async_copy(wtab_sh.at[li], wv, sem).wait()

    def _san(i, carry):
        sl = pl.ds(i * 16, 16)
        ok = li[sl] < LTRASH
        wv[sl] = jnp.where(ok, wv[sl], jnp.zeros((16,), jnp.int32))
        return carry

    lax.fori_loop(0, R_CH16, _san, 0)

    pltpu.sync_copy(wv, out_hbm.at[cid].at[pl.ds(q0, R_CHUNK)])


_R = 512


def _dense_body(ms_ref, md_ref, tb_ref, w_ref, b_ref, out_ref):
    ms = ms_ref[...].astype(jnp.float32)
    md = md_ref[...].astype(jnp.float32)
    w = w_ref[...]
    bstk = b_ref[...]

    def mm(x, k):
        return jnp.dot(x, w[k], preferred_element_type=jnp.float32)

    def bias(k):
        return bstk[k:k + 1, :]

    tf = jnp.cos(tb_ref[...] * bias(0) + bias(1))
    t_c = mm(tf, 2) + bias(2)
    msg_s = mm(ms, 0) + mm(md, 1) + t_c
    msg_d = mm(md, 0) + mm(ms, 1) + t_c

    def sig(x):
        return 1.0 / (1.0 + jnp.exp(-x))

    def gru(msg, h):
        r = sig(mm(msg, 3) + bias(3) + mm(h, 6) + bias(6))
        z = sig(mm(msg, 4) + bias(4) + mm(h, 7) + bias(7))
        n = jnp.tanh(mm(msg, 5) + bias(5) + r * (mm(h, 8) + bias(8)))
        return (1.0 - z) * n + z * h

    new_s = gru(msg_s, ms)
    new_d = gru(msg_d, md)
    out_ref[0] = jnp.maximum(mm(new_s, 9) + bias(9), 0.0)
    out_ref[1] = jnp.maximum(mm(new_d, 9) + bias(9), 0.0)


def _tc_dense(ms2, md2, tb2, wstk, bstk):
    grid = B4 // _R
    return pl.pallas_call(
        _dense_body,
        grid=(grid,),
        in_specs=[
            pl.BlockSpec((_R, 128), lambda i: (i, 0)),
            pl.BlockSpec((_R, 128), lambda i: (i, 0)),
            pl.BlockSpec((_R, 128), lambda i: (i, 0)),
            pl.BlockSpec((11, 128, 128), lambda i: (0, 0, 0)),
            pl.BlockSpec((16, 128), lambda i: (0, 0)),
        ],
        out_specs=pl.BlockSpec((2, _R, 128), lambda i: (0, i, 0)),
        out_shape=jax.ShapeDtypeStruct((2, B4, 128), jnp.float32),
    )(ms2, md2, tb2, wstk, bstk)


def _head_body(xs_ref, xd_ref, w_ref, b_ref, out_ref):
    diff = jnp.abs(xs_ref[...] - xd_ref[...])
    out_ref[...] = (
        jnp.dot(diff, w_ref[...][10], preferred_element_type=jnp.float32)
        + b_ref[...][10:11, :])


def _tc_head(xs2, xd2, wstk, bstk):
    grid = B4 // _R
    return pl.pallas_call(
        _head_body,
        grid=(grid,),
        in_specs=[
            pl.BlockSpec((_R, 128), lambda i: (i, 0)),
            pl.BlockSpec((_R, 128), lambda i: (i, 0)),
            pl.BlockSpec((11, 128, 128), lambda i: (0, 0, 0)),
            pl.BlockSpec((16, 128), lambda i: (0, 0)),
        ],
        out_specs=pl.BlockSpec((_R, 128), lambda i: (i, 0)),
        out_shape=jax.ShapeDtypeStruct((B4, 128), jnp.float32),
    )(xs2, xd2, wstk, bstk)


def kernel(src, dst, t, memory, w_t, b_t, W_msg, b_msg, W_ih, W_hh, b_ih, b_hh,
           W_emb, b_emb, W_lin, b_lin):
    cat_flat = jnp.concatenate([src, dst]).astype(jnp.int32)
    cat_idx2d = cat_flat.reshape(TWOB // 128, 128)
    jarr = jnp.arange(TWOB, dtype=jnp.int32)

    ws = jnp.stack([
        W_msg[:, :32].T, W_msg[:, 32:64].T, W_msg[:, 64:].T,
        W_ih[:32, :].T, W_ih[32:64, :].T, W_ih[64:, :].T,
        W_hh[:32, :].T, W_hh[32:64, :].T, W_hh[64:, :].T,
        W_emb.T, W_lin.T,
    ]).astype(jnp.float32)
    eye4 = jnp.eye(4, dtype=jnp.float32)
    wstk = (eye4[None, :, None, :, None] * ws[:, None, :, None, :]).reshape(11, 128, 128)

    def t4(v):
        return jnp.tile(v.astype(jnp.float32), 4)

    bstk = jnp.concatenate([
        jnp.stack([t4(w_t), t4(b_t), t4(b_msg),
                   t4(b_ih[:32]), t4(b_ih[32:64]), t4(b_ih[64:]),
                   t4(b_hh[:32]), t4(b_hh[32:64]), t4(b_hh[64:]),
                   t4(b_emb), t4(b_lin)]),
        jnp.zeros((5, 128), jnp.float32),
    ])

    mem_cat = _make_sc_gather(jnp.bfloat16)(memory.astype(jnp.bfloat16), cat_idx2d)
    ms2 = mem_cat[:NEV].reshape(B4, 128)
    md2 = mem_cat[NEV:].reshape(B4, 128)
    tb2 = jnp.repeat(t.astype(jnp.float32), MEM).reshape(B4, 128)

    ecat = _tc_dense(ms2, md2, tb2, wstk, bstk).reshape(TWOB, MEM)

    gvs = _make_sc_resolve()(cat_flat, jarr)
    gv2d = (gvs[0] + gvs[1]).reshape(TWOB // 128, 128)
    xcat = _make_sc_gather(jnp.float32)(ecat, gv2d)
    xs2 = xcat[:NEV].reshape(B4, 128)
    xd2 = xcat[NEV:].reshape(B4, 128)

    out2 = _tc_head(xs2, xd2, wstk, bstk)
    return out2.reshape(NEV, MEM)


# speedup vs baseline: 1.2963x; 1.2963x over previous
import functools

import jax
import jax.numpy as jnp
from jax import lax
from jax.experimental import pallas as pl
from jax.experimental.pallas import tpu as pltpu
from jax.experimental.pallas import tpu_sc as plsc

NODE = 1_000_000
MEM = 32
NEV = 16384
TWOB = 2 * NEV
B4 = NEV // 4

NC, NS = 2, 16
NW = NC * NS
G_ROWS = TWOB // NW // 128
R_ROWS = TWOB // NS // 128
R_CHUNK = R_ROWS * 128
WTAB = NODE + 128
TRASH = NODE
ROUNDS = 3

_SC_PARAMS = pltpu.CompilerParams(use_tc_tiling_on_sc=False)


def _sc_mesh():
    return plsc.VectorSubcoreMesh(
        core_axis_name="c", subcore_axis_name="s", num_cores=NC, num_subcores=NS)


@functools.cache
def _make_sc_gather():
    return pl.kernel(
        _sc_gather_body,
        out_type=jax.ShapeDtypeStruct((TWOB, MEM), jnp.float32),
        mesh=_sc_mesh(),
        scratch_types=[
            pltpu.VMEM((G_ROWS, 128), jnp.int32),
            pltpu.VMEM((G_ROWS * 128, MEM), jnp.float32),
            pltpu.SemaphoreType.DMA,
        ],
        compiler_params=_SC_PARAMS,
    )


def _sc_gather_body(mem_hbm, idx_hbm, out_hbm, idx_v, rows_v, sem):
    wid = lax.axis_index("s") * NC + lax.axis_index("c")
    r0 = wid * G_ROWS
    pltpu.sync_copy(idx_hbm.at[pl.ds(r0, G_ROWS)], idx_v)
    cps = [
        pltpu.async_copy(mem_hbm.at[idx_v.at[b]], rows_v.at[pl.ds(b * 128, 128)], sem)
        for b in range(G_ROWS)
    ]
    for c in cps:
        c.wait()
    pltpu.sync_copy(rows_v, out_hbm.at[pl.ds(r0 * 128, G_ROWS * 128)])


NSPLIT = NODE // 2
LTRASH = NSPLIT
LTAB = NSPLIT + 128
R_CH16 = R_CHUNK // 16


@functools.cache
def _make_sc_resolve():
    return pl.kernel(
        _sc_resolve_body,
        out_type=jax.ShapeDtypeStruct((NC, TWOB), jnp.int32),
        mesh=_sc_mesh(),
        scratch_types=[
            pltpu.VMEM_SHARED((LTAB,), jnp.int32),
            pltpu.VMEM((R_CHUNK,), jnp.int32),
            pltpu.VMEM((R_CHUNK,), jnp.int32),
            pltpu.VMEM((R_CHUNK,), jnp.int32),
            pltpu.VMEM((R_CHUNK,), jnp.int32),
            pltpu.SemaphoreType.DMA,
        ],
        compiler_params=_SC_PARAMS,
    )


def _sc_resolve_body(idx_hbm, j_hbm, out_hbm, wtab_sh,
                     li, jv, wv, tv, sem):
    cid = lax.axis_index("c")
    sid = lax.axis_index("s")
    base = cid * NSPLIT
    q0 = sid * R_CHUNK
    pltpu.sync_copy(idx_hbm.at[pl.ds(q0, R_CHUNK)], li)
    pltpu.sync_copy(j_hbm.at[pl.ds(q0, R_CHUNK)], jv)

    def _localize(i, carry):
        sl = pl.ds(i * 16, 16)
        loc = li[sl] - base
        ok = (loc >= 0) & (loc < NSPLIT)
        li[sl] = jnp.where(ok, loc, jnp.full((16,), LTRASH, jnp.int32))
        return carry

    lax.fori_loop(0, R_CH16, _localize, 0)

    pltpu.async_copy(jv, wtab_sh.at[li], sem).wait()
    plsc.subcore_barrier()

    for _ in range(ROUNDS):
        pltpu.async_copy(wtab_sh.at[li], wv, sem).wait()
        plsc.subcore_barrier()

        def _row(i, carry):
            sl = pl.ds(i * 16, 16)
            keep = (jv[sl] > wv[sl]) & (li[sl] < LTRASH)
            tv[sl] = jnp.where(keep, li[sl], jnp.full((16,), LTRASH, jnp.int32))
            return carry

        lax.fori_loop(0, R_CH16, _row, 0)

        pltpu.async_copy(jv, wtab_sh.at[tv], sem).wait()
        plsc.subcore_barrier()

    pltpu.async_copy(wtab_sh.at[li], wv, sem).wait()

    def _san(i, carry):
        sl = pl.ds(i * 16, 16)
        ok = li[sl] < LTRASH
        wv[sl] = jnp.where(ok, wv[sl], jnp.zeros((16,), jnp.int32))
        return carry

    lax.fori_loop(0, R_CH16, _san, 0)

    pltpu.sync_copy(wv, out_hbm.at[cid].at[pl.ds(q0, R_CHUNK)])


_R = 512


def _dense_body(ms_ref, md_ref, tb_ref, w_ref, b_ref, out_ref):
    ms = ms_ref[...]
    md = md_ref[...]
    w = w_ref[...]
    bstk = b_ref[...]

    def mm(x, k):
        return jnp.dot(x, w[k], preferred_element_type=jnp.float32)

    def bias(k):
        return bstk[k:k + 1, :]

    tf = jnp.cos(tb_ref[...] * bias(0) + bias(1))
    t_c = mm(tf, 2) + bias(2)
    msg_s = mm(ms, 0) + mm(md, 1) + t_c
    msg_d = mm(md, 0) + mm(ms, 1) + t_c

    def sig(x):
        return 1.0 / (1.0 + jnp.exp(-x))

    def gru(msg, h):
        r = sig(mm(msg, 3) + bias(3) + mm(h, 6) + bias(6))
        z = sig(mm(msg, 4) + bias(4) + mm(h, 7) + bias(7))
        n = jnp.tanh(mm(msg, 5) + bias(5) + r * (mm(h, 8) + bias(8)))
        return (1.0 - z) * n + z * h

    new_s = gru(msg_s, ms)
    new_d = gru(msg_d, md)
    out_ref[0] = jnp.maximum(mm(new_s, 9) + bias(9), 0.0)
    out_ref[1] = jnp.maximum(mm(new_d, 9) + bias(9), 0.0)


def _tc_dense(ms2, md2, tb2, wstk, bstk):
    grid = B4 // _R
    return pl.pallas_call(
        _dense_body,
        grid=(grid,),
        in_specs=[
            pl.BlockSpec((_R, 128), lambda i: (i, 0)),
            pl.BlockSpec((_R, 128), lambda i: (i, 0)),
            pl.BlockSpec((_R, 128), lambda i: (i, 0)),
            pl.BlockSpec((11, 128, 128), lambda i: (0, 0, 0)),
            pl.BlockSpec((16, 128), lambda i: (0, 0)),
        ],
        out_specs=pl.BlockSpec((2, _R, 128), lambda i: (0, i, 0)),
        out_shape=jax.ShapeDtypeStruct((2, B4, 128), jnp.float32),
    )(ms2, md2, tb2, wstk, bstk)


def _head_body(xs_ref, xd_ref, w_ref, b_ref, out_ref):
    diff = jnp.abs(xs_ref[...] - xd_ref[...])
    out_ref[...] = (
        jnp.dot(diff, w_ref[...][10], preferred_element_type=jnp.float32)
        + b_ref[...][10:11, :])


def _tc_head(xs2, xd2, wstk, bstk):
    grid = B4 // _R
    return pl.pallas_call(
        _head_body,
        grid=(grid,),
        in_specs=[
            pl.BlockSpec((_R, 128), lambda i: (i, 0)),
            pl.BlockSpec((_R, 128), lambda i: (i, 0)),
            pl.BlockSpec((11, 128, 128), lambda i: (0, 0, 0)),
            pl.BlockSpec((16, 128), lambda i: (0, 0)),
        ],
        out_specs=pl.BlockSpec((_R, 128), lambda i: (i, 0)),
        out_shape=jax.ShapeDtypeStruct((B4, 128), jnp.float32),
    )(xs2, xd2, wstk, bstk)


def kernel(src, dst, t, memory, w_t, b_t, W_msg, b_msg, W_ih, W_hh, b_ih, b_hh,
           W_emb, b_emb, W_lin, b_lin):
    cat_flat = jnp.concatenate([src, dst]).astype(jnp.int32)
    cat_idx2d = cat_flat.reshape(TWOB // 128, 128)
    jarr = jnp.arange(TWOB, dtype=jnp.int32)

    ws = jnp.stack([
        W_msg[:, :32].T, W_msg[:, 32:64].T, W_msg[:, 64:].T,
        W_ih[:32, :].T, W_ih[32:64, :].T, W_ih[64:, :].T,
        W_hh[:32, :].T, W_hh[32:64, :].T, W_hh[64:, :].T,
        W_emb.T, W_lin.T,
    ]).astype(jnp.float32)
    eye4 = jnp.eye(4, dtype=jnp.float32)
    wstk = (eye4[None, :, None, :, None] * ws[:, None, :, None, :]).reshape(11, 128, 128)

    def t4(v):
        return jnp.tile(v.astype(jnp.float32), 4)

    bstk = jnp.concatenate([
        jnp.stack([t4(w_t), t4(b_t), t4(b_msg),
                   t4(b_ih[:32]), t4(b_ih[32:64]), t4(b_ih[64:]),
                   t4(b_hh[:32]), t4(b_hh[32:64]), t4(b_hh[64:]),
                   t4(b_emb), t4(b_lin)]),
        jnp.zeros((5, 128), jnp.float32),
    ])

    mem_cat = _make_sc_gather()(memory.astype(jnp.float32), cat_idx2d)
    ms2 = mem_cat[:NEV].reshape(B4, 128)
    md2 = mem_cat[NEV:].reshape(B4, 128)
    tb2 = jnp.repeat(t.astype(jnp.float32), MEM).reshape(B4, 128)

    ecat = _tc_dense(ms2, md2, tb2, wstk, bstk).reshape(TWOB, MEM)

    gvs = _make_sc_resolve()(cat_flat, jarr)
    gv2d = (gvs[0] + gvs[1]).reshape(TWOB // 128, 128)
    xcat = _make_sc_gather()(ecat, gv2d)
    xs2 = xcat[:NEV].reshape(B4, 128)
    xd2 = xcat[NEV:].reshape(B4, 128)

    out2 = _tc_head(xs2, xd2, wstk, bstk)
    return out2.reshape(NEV, MEM)
